# baseline (speedup 1.0000x reference)
import functools

import jax
import jax.numpy as jnp
from jax import lax
from jax.experimental import pallas as pl
from jax.experimental.pallas import tpu as pltpu
from jax.experimental.pallas import tpu_sc as plsc

B = 16384
L = 50
DF = 128
H = 64
NC, NS = 2, 16
NW = NC * NS
BPW = B // NW
CB = 8
NCHUNK = BPW // CB
IPC = CB * L
GW = 80
NG = IPC // GW
VL = 16
NH = H // VL
VOCAB_QUADS = 250000


def _sc_bag(title_flat, emb_text):
    mesh = plsc.VectorSubcoreMesh(core_axis_name="c", subcore_axis_name="s")
    ILV = plsc.PackFormat.INTERLEAVED

    @functools.partial(
        pl.kernel,
        out_type=jax.ShapeDtypeStruct((B, H), jnp.bfloat16),
        mesh=mesh,
        compiler_params=pltpu.CompilerParams(use_tc_tiling_on_sc=False,
                                             needs_layout_passes=False),
        scratch_types=[
            pltpu.VMEM((IPC,), jnp.int32),
            pltpu.VMEM((IPC, H), jnp.bfloat16),
            pltpu.VMEM((CB, H), jnp.bfloat16),
            pltpu.SemaphoreType.DMA,
        ],
    )
    def k(title_hbm, etext_hbm, tsum_hbm, idx_v, rows_v, out_v, sem):
        wid = lax.axis_index("s") * NC + lax.axis_index("c")

        def unp(r, u):
            return plsc.unpack(rows_v[r, pl.ds(u * 32, 32)], format=ILV)

        def chunk_body(c, carry):
            b0 = wid * BPW + c * CB
            pltpu.sync_copy(title_hbm.at[pl.ds(b0 * L, IPC)], idx_v)
            copies = [
                pltpu.async_copy(etext_hbm.at[idx_v.at[pl.ds(g * GW, GW)]],
                                 rows_v.at[pl.ds(g * GW, GW), :], sem)
                for g in range(NG)
            ]
            for cp in copies:
                cp.wait()

            for b in range(CB):
                r0 = b * L
                acc = list(unp(r0, 0) + unp(r0, 1))
                for j in range(1, L):
                    x = unp(r0 + j, 0) + unp(r0 + j, 1)
                    for q in range(4):
                        acc[q] = acc[q] + x[q]
                out_v[b, pl.ds(0, 32)] = plsc.pack(acc[0], acc[1], format=ILV)
                out_v[b, pl.ds(32, 32)] = plsc.pack(acc[2], acc[3], format=ILV)
            pltpu.sync_copy(out_v, tsum_hbm.at[pl.ds(b0, CB), :])
            return carry

        lax.fori_loop(0, NCHUNK, chunk_body, 0)

    return k(title_flat, emb_text)


def _sc_cat(cat_ids, emb_cat):
    NBUF = 8
    NGRP = BPW // NBUF

    mesh = plsc.VectorSubcoreMesh(core_axis_name="c", subcore_axis_name="s")

    @functools.partial(
        pl.kernel,
        out_type=jax.ShapeDtypeStruct((B, H), jnp.float32),
        mesh=mesh,
        scratch_types=[
            pltpu.VMEM((BPW,), jnp.int32),
            pltpu.VMEM((2, NBUF, 8, H), jnp.float32),
            pltpu.VMEM((NBUF, H), jnp.float32),
            pltpu.SemaphoreType.DMA,
            [pltpu.SemaphoreType.DMA] * (2 * NBUF),
        ],
    )
    def k(cat_hbm, ecat_hbm, crow_hbm, idx_v, tiles_v, out_v, isem, sems):
        wid = lax.axis_index("s") * NC + lax.axis_index("c")
        i0 = wid * BPW
        pltpu.async_copy(cat_hbm.at[pl.ds(i0, BPW)], idx_v, isem).wait()

        def load_ids(t):
            return idx_v[pl.ds(t * 2 * NBUF, 2 * NBUF)]

        def fire_group(ids, bank):
            for p in range(NBUF):
                tid = (ids[bank * NBUF + p] // 8) * 8
                pltpu.async_copy(
                    ecat_hbm.at[pl.ds(pl.multiple_of(tid, 8), 8), :],
                    tiles_v.at[bank, p], sems[bank * NBUF + p])

        def drain_group(ids, g, bank):
            for p in range(NBUF):
                pltpu.make_async_copy(ecat_hbm.at[pl.ds(0, 8), :],
                                      tiles_v.at[bank, p],
                                      sems[bank * NBUF + p]).wait()
                r = lax.rem(ids[bank * NBUF + p], 8)
                for h in range(NH):
                    out_v[p, pl.ds(h * VL, VL)] = \
                        tiles_v[bank, p, r, pl.ds(h * VL, VL)]
            pltpu.sync_copy(out_v,
                            crow_hbm.at[pl.ds(i0 + g * NBUF, NBUF), :])

        fire_group(load_ids(0), 0)

        def body(t, _):
            g = 2 * t
            ids = load_ids(t)
            fire_group(ids, 1)
            drain_group(ids, g, 0)

            @pl.when(t + 1 < NGRP // 2)
            def _():
                fire_group(load_ids(t + 1), 0)

            drain_group(ids, g + 1, 1)
            return 0

        lax.fori_loop(0, NGRP // 2, body, 0)

    return k(cat_ids, emb_cat)


TB = 512
NTB = (1000001 + TB - 1) // TB


def _tc_transpose_table(ecat_t, ident):
    def body(t_ref, i_ref, o_ref):
        blk = t_ref[...].astype(jnp.bfloat16)
        o_ref[...] = lax.dot_general(blk, i_ref[...],
                                     (((0,), (0,)), ((), ())),
                                     preferred_element_type=jnp.float32)

    return pl.pallas_call(
        body,
        grid=(NTB,),
        in_specs=[
            pl.BlockSpec((H, TB), lambda i: (0, i)),
            pl.BlockSpec((H, H), lambda i: (0, 0)),
        ],
        out_specs=pl.BlockSpec((TB, H), lambda i: (i, 0)),
        out_shape=jax.ShapeDtypeStruct((NTB * TB, H), jnp.float32),
    )(ecat_t, ident)


def _tc_combine(float_feat, W, b_row, len_col, tsum, crow2, cid_col):
    BT = 2048

    def body(ff_ref, w_ref, b_ref, len_ref, ts_ref, cr_ref, cid_ref, o_ref):
        inv = 1.0 / len_ref[...].astype(jnp.float32)
        proj = jnp.dot(ff_ref[...], w_ref[...],
                       preferred_element_type=jnp.float32)
        cat = cr_ref[...]
        o_ref[...] = (cat + proj + b_ref[...]
                      + ts_ref[...].astype(jnp.float32) * inv)

    return pl.pallas_call(
        body,
        grid=(B // BT,),
        in_specs=[
            pl.BlockSpec((BT, DF), lambda i: (i, 0)),
            pl.BlockSpec((DF, H), lambda i: (0, 0)),
            pl.BlockSpec((1, H), lambda i: (0, 0)),
            pl.BlockSpec((BT, 1), lambda i: (i, 0)),
            pl.BlockSpec((BT, H), lambda i: (i, 0)),
            pl.BlockSpec((BT, H), lambda i: (i, 0)),
            pl.BlockSpec((BT, 1), lambda i: (i, 0)),
        ],
        out_specs=pl.BlockSpec((BT, H), lambda i: (i, 0)),
        out_shape=jax.ShapeDtypeStruct((B, H), jnp.float32),
    )(float_feat, W, b_row, len_col, tsum, crow2, cid_col)


def kernel(cat_feat, float_feat, title, title_len, emb_cat, W_float, b_float,
           emb_text):
    title_flat = title.astype(jnp.int32).reshape(-1)
    cat_ids = cat_feat.astype(jnp.int32)
    ecat_rows = _tc_transpose_table(emb_cat.T,
                                    jnp.eye(H, dtype=jnp.bfloat16))
    tsum = _sc_bag(title_flat, emb_text.astype(jnp.bfloat16))
    crow2 = _sc_cat(cat_ids, ecat_rows)
    return _tc_combine(float_feat, W_float, b_float.reshape(1, H),
                       title_len.astype(jnp.int32).reshape(B, 1), tsum,
                       crow2, cat_ids.reshape(B, 1))


# speedup vs baseline: 2.7702x; 2.7702x over previous
import functools

import jax
import jax.numpy as jnp
from jax import lax
from jax.experimental import pallas as pl
from jax.experimental.pallas import tpu as pltpu
from jax.experimental.pallas import tpu_sc as plsc

B = 16384
L = 50
DF = 128
H = 64
NC, NS = 2, 16
NW = NC * NS
BPW = B // NW
CB = 8
NCHUNK = BPW // CB
IPC = CB * L
GW = 80
NG = IPC // GW
VL = 16
NH = H // VL
VOCAB_QUADS = 250000


def _sc_bag(title_flat, emb_text):
    mesh = plsc.VectorSubcoreMesh(core_axis_name="c", subcore_axis_name="s")
    ILV = plsc.PackFormat.INTERLEAVED

    @functools.partial(
        pl.kernel,
        out_type=jax.ShapeDtypeStruct((B, H), jnp.bfloat16),
        mesh=mesh,
        compiler_params=pltpu.CompilerParams(use_tc_tiling_on_sc=False,
                                             needs_layout_passes=False),
        scratch_types=[
            pltpu.VMEM((IPC,), jnp.int32),
            pltpu.VMEM((IPC, H), jnp.bfloat16),
            pltpu.VMEM((CB, H), jnp.bfloat16),
            pltpu.SemaphoreType.DMA,
        ],
    )
    def k(title_hbm, etext_hbm, tsum_hbm, idx_v, rows_v, out_v, sem):
        wid = lax.axis_index("s") * NC + lax.axis_index("c")

        def unp(r, u):
            return plsc.unpack(rows_v[r, pl.ds(u * 32, 32)], format=ILV)

        def chunk_body(c, carry):
            b0 = wid * BPW + c * CB
            pltpu.sync_copy(title_hbm.at[pl.ds(b0 * L, IPC)], idx_v)
            copies = [
                pltpu.async_copy(etext_hbm.at[idx_v.at[pl.ds(g * GW, GW)]],
                                 rows_v.at[pl.ds(g * GW, GW), :], sem)
                for g in range(NG)
            ]
            for cp in copies:
                cp.wait()

            for b in range(CB):
                r0 = b * L
                acc = list(unp(r0, 0) + unp(r0, 1))
                for j in range(1, L):
                    x = unp(r0 + j, 0) + unp(r0 + j, 1)
                    for q in range(4):
                        acc[q] = acc[q] + x[q]
                out_v[b, pl.ds(0, 32)] = plsc.pack(acc[0], acc[1], format=ILV)
                out_v[b, pl.ds(32, 32)] = plsc.pack(acc[2], acc[3], format=ILV)
            pltpu.sync_copy(out_v, tsum_hbm.at[pl.ds(b0, CB), :])
            return carry

        lax.fori_loop(0, NCHUNK, chunk_body, 0)

    return k(title_flat, emb_text)


def _sc_cat(cat_ids, emb_cat):
    NBUF = 8
    NGRP = BPW // NBUF

    mesh = plsc.VectorSubcoreMesh(core_axis_name="c", subcore_axis_name="s")

    @functools.partial(
        pl.kernel,
        out_type=jax.ShapeDtypeStruct((B, H), jnp.float32),
        mesh=mesh,
        scratch_types=[
            pltpu.VMEM((BPW,), jnp.int32),
            pltpu.VMEM((2, NBUF, 8, H), jnp.float32),
            pltpu.VMEM((NBUF, H), jnp.float32),
            pltpu.SemaphoreType.DMA,
            [pltpu.SemaphoreType.DMA] * (2 * NBUF),
        ],
    )
    def k(cat_hbm, ecat_hbm, crow_hbm, idx_v, tiles_v, out_v, isem, sems):
        wid = lax.axis_index("s") * NC + lax.axis_index("c")
        i0 = wid * BPW
        pltpu.async_copy(cat_hbm.at[pl.ds(i0, BPW)], idx_v, isem).wait()

        def load_ids(t):
            return idx_v[pl.ds(t * 2 * NBUF, 2 * NBUF)]

        def fire_group(ids, bank):
            for p in range(NBUF):
                tid = (ids[bank * NBUF + p] // 8) * 8
                pltpu.async_copy(
                    ecat_hbm.at[pl.ds(pl.multiple_of(tid, 8), 8), :],
                    tiles_v.at[bank, p], sems[bank * NBUF + p])

        def drain_group(ids, g, bank):
            for p in range(NBUF):
                pltpu.make_async_copy(ecat_hbm.at[pl.ds(0, 8), :],
                                      tiles_v.at[bank, p],
                                      sems[bank * NBUF + p]).wait()
                r = lax.rem(ids[bank * NBUF + p], 8)
                for h in range(NH):
                    out_v[p, pl.ds(h * VL, VL)] = \
                        tiles_v[bank, p, r, pl.ds(h * VL, VL)]
            pltpu.sync_copy(out_v,
                            crow_hbm.at[pl.ds(i0 + g * NBUF, NBUF), :])

        fire_group(load_ids(0), 0)

        def body(t, _):
            g = 2 * t
            ids = load_ids(t)
            fire_group(ids, 1)
            drain_group(ids, g, 0)

            @pl.when(t + 1 < NGRP // 2)
            def _():
                fire_group(load_ids(t + 1), 0)

            drain_group(ids, g + 1, 1)
            return 0

        lax.fori_loop(0, NGRP // 2, body, 0)

    return k(cat_ids, emb_cat)


TB = 4096
NTB = (1000001 + TB - 1) // TB


def _tc_transpose_table(ecat_t, ident):
    def body(t_ref, i_ref, o_ref):
        blk = t_ref[...].astype(jnp.bfloat16)
        o_ref[...] = lax.dot_general(blk, i_ref[...],
                                     (((0,), (0,)), ((), ())),
                                     preferred_element_type=jnp.float32)

    return pl.pallas_call(
        body,
        grid=(NTB,),
        in_specs=[
            pl.BlockSpec((H, TB), lambda i: (0, i)),
            pl.BlockSpec((H, H), lambda i: (0, 0)),
        ],
        out_specs=pl.BlockSpec((TB, H), lambda i: (i, 0)),
        out_shape=jax.ShapeDtypeStruct((NTB * TB, H), jnp.float32),
    )(ecat_t, ident)


def _tc_combine(float_feat, W, b_row, len_col, tsum, crow2, cid_col):
    BT = 2048

    def body(ff_ref, w_ref, b_ref, len_ref, ts_ref, cr_ref, cid_ref, o_ref):
        inv = 1.0 / len_ref[...].astype(jnp.float32)
        proj = jnp.dot(ff_ref[...], w_ref[...],
                       preferred_element_type=jnp.float32)
        cat = cr_ref[...]
        o_ref[...] = (cat + proj + b_ref[...]
                      + ts_ref[...].astype(jnp.float32) * inv)

    return pl.pallas_call(
        body,
        grid=(B // BT,),
        in_specs=[
            pl.BlockSpec((BT, DF), lambda i: (i, 0)),
            pl.BlockSpec((DF, H), lambda i: (0, 0)),
            pl.BlockSpec((1, H), lambda i: (0, 0)),
            pl.BlockSpec((BT, 1), lambda i: (i, 0)),
            pl.BlockSpec((BT, H), lambda i: (i, 0)),
            pl.BlockSpec((BT, H), lambda i: (i, 0)),
            pl.BlockSpec((BT, 1), lambda i: (i, 0)),
        ],
        out_specs=pl.BlockSpec((BT, H), lambda i: (i, 0)),
        out_shape=jax.ShapeDtypeStruct((B, H), jnp.float32),
    )(float_feat, W, b_row, len_col, tsum, crow2, cid_col)


def kernel(cat_feat, float_feat, title, title_len, emb_cat, W_float, b_float,
           emb_text):
    title_flat = title.astype(jnp.int32).reshape(-1)
    cat_ids = cat_feat.astype(jnp.int32)
    ecat_rows = _tc_transpose_table(emb_cat.T,
                                    jnp.eye(H, dtype=jnp.bfloat16))
    tsum = _sc_bag(title_flat, emb_text.astype(jnp.bfloat16))
    crow2 = _sc_cat(cat_ids, ecat_rows)
    return _tc_combine(float_feat, W_float, b_float.reshape(1, H),
                       title_len.astype(jnp.int32).reshape(B, 1), tsum,
                       crow2, cat_ids.reshape(B, 1))


# speedup vs baseline: 2.9884x; 1.0787x over previous
import functools

import jax
import jax.numpy as jnp
from jax import lax
from jax.experimental import pallas as pl
from jax.experimental.pallas import tpu as pltpu
from jax.experimental.pallas import tpu_sc as plsc

B = 16384
L = 50
DF = 128
H = 64
NC, NS = 2, 16
NW = NC * NS
BPW = B // NW
CB = 8
NCHUNK = BPW // CB
IPC = CB * L
GW = 80
NG = IPC // GW
VL = 16
NH = H // VL
VOCAB_QUADS = 250000


def _sc_bag(title_flat, emb_text):
    mesh = plsc.VectorSubcoreMesh(core_axis_name="c", subcore_axis_name="s")
    ILV = plsc.PackFormat.INTERLEAVED

    @functools.partial(
        pl.kernel,
        out_type=jax.ShapeDtypeStruct((B, H), jnp.bfloat16),
        mesh=mesh,
        compiler_params=pltpu.CompilerParams(use_tc_tiling_on_sc=False,
                                             needs_layout_passes=False),
        scratch_types=[
            pltpu.VMEM((IPC,), jnp.int32),
            pltpu.VMEM((IPC, H), jnp.bfloat16),
            pltpu.VMEM((CB, H), jnp.bfloat16),
            pltpu.SemaphoreType.DMA,
        ],
    )
    def k(title_hbm, etext_hbm, tsum_hbm, idx_v, rows_v, out_v, sem):
        wid = lax.axis_index("s") * NC + lax.axis_index("c")

        def unp(r, u):
            return plsc.unpack(rows_v[r, pl.ds(u * 32, 32)], format=ILV)

        def chunk_body(c, carry):
            b0 = wid * BPW + c * CB
            pltpu.sync_copy(title_hbm.at[pl.ds(b0 * L, IPC)], idx_v)
            copies = [
                pltpu.async_copy(etext_hbm.at[idx_v.at[pl.ds(g * GW, GW)]],
                                 rows_v.at[pl.ds(g * GW, GW), :], sem)
                for g in range(NG)
            ]
            for cp in copies:
                cp.wait()

            for b in range(CB):
                r0 = b * L
                acc = list(unp(r0, 0) + unp(r0, 1))
                for j in range(1, L):
                    x = unp(r0 + j, 0) + unp(r0 + j, 1)
                    for q in range(4):
                        acc[q] = acc[q] + x[q]
                out_v[b, pl.ds(0, 32)] = plsc.pack(acc[0], acc[1], format=ILV)
                out_v[b, pl.ds(32, 32)] = plsc.pack(acc[2], acc[3], format=ILV)
            pltpu.sync_copy(out_v, tsum_hbm.at[pl.ds(b0, CB), :])
            return carry

        lax.fori_loop(0, NCHUNK, chunk_body, 0)

    return k(title_flat, emb_text)


def _sc_cat(cat_ids, emb_cat):
    NBUF = 8
    NGRP = BPW // NBUF

    mesh = plsc.VectorSubcoreMesh(core_axis_name="c", subcore_axis_name="s")

    @functools.partial(
        pl.kernel,
        out_type=jax.ShapeDtypeStruct((B, H), jnp.float32),
        mesh=mesh,
        scratch_types=[
            pltpu.VMEM((BPW,), jnp.int32),
            pltpu.VMEM((2, NBUF, 8, H), jnp.float32),
            pltpu.VMEM((NBUF, H), jnp.float32),
            pltpu.SemaphoreType.DMA,
            [pltpu.SemaphoreType.DMA] * (2 * NBUF),
        ],
    )
    def k(cat_hbm, ecat_hbm, crow_hbm, idx_v, tiles_v, out_v, isem, sems):
        wid = lax.axis_index("s") * NC + lax.axis_index("c")
        i0 = wid * BPW
        pltpu.async_copy(cat_hbm.at[pl.ds(i0, BPW)], idx_v, isem).wait()

        def load_ids(t):
            return idx_v[pl.ds(t * 2 * NBUF, 2 * NBUF)]

        def fire_group(ids, bank):
            for p in range(NBUF):
                tid = (ids[bank * NBUF + p] // 8) * 8
                pltpu.async_copy(
                    ecat_hbm.at[pl.ds(pl.multiple_of(tid, 8), 8), :],
                    tiles_v.at[bank, p], sems[bank * NBUF + p])

        def drain_group(ids, g, bank):
            for p in range(NBUF):
                pltpu.make_async_copy(ecat_hbm.at[pl.ds(0, 8), :],
                                      tiles_v.at[bank, p],
                                      sems[bank * NBUF + p]).wait()
                r = lax.rem(ids[bank * NBUF + p], 8)
                for h in range(NH):
                    out_v[p, pl.ds(h * VL, VL)] = \
                        tiles_v[bank, p, r, pl.ds(h * VL, VL)]
            pltpu.sync_copy(out_v,
                            crow_hbm.at[pl.ds(i0 + g * NBUF, NBUF), :])

        fire_group(load_ids(0), 0)

        def body(t, _):
            g = 2 * t
            ids = load_ids(t)
            fire_group(ids, 1)
            drain_group(ids, g, 0)

            @pl.when(t + 1 < NGRP // 2)
            def _():
                fire_group(load_ids(t + 1), 0)

            drain_group(ids, g + 1, 1)
            return 0

        lax.fori_loop(0, NGRP // 2, body, 0)

    return k(cat_ids, emb_cat)


TB = 8192
NTB = (1000001 + TB - 1) // TB


def _tc_transpose_table(ecat_t, ident):
    def body(t_ref, i_ref, o_ref):
        blk = t_ref[...].astype(jnp.bfloat16)
        o_ref[...] = lax.dot_general(blk, i_ref[...],
                                     (((0,), (0,)), ((), ())),
                                     preferred_element_type=jnp.float32)

    return pl.pallas_call(
        body,
        grid=(NTB,),
        in_specs=[
            pl.BlockSpec((H, TB), lambda i: (0, i)),
            pl.BlockSpec((H, H), lambda i: (0, 0)),
        ],
        out_specs=pl.BlockSpec((TB, H), lambda i: (i, 0)),
        out_shape=jax.ShapeDtypeStruct((NTB * TB, H), jnp.float32),
    )(ecat_t, ident)


def _tc_combine(float_feat, W, b_row, len_col, tsum, crow2, cid_col):
    BT = 2048

    def body(ff_ref, w_ref, b_ref, len_ref, ts_ref, cr_ref, cid_ref, o_ref):
        inv = 1.0 / len_ref[...].astype(jnp.float32)
        proj = jnp.dot(ff_ref[...], w_ref[...],
                       preferred_element_type=jnp.float32)
        cat = cr_ref[...]
        o_ref[...] = (cat + proj + b_ref[...]
                      + ts_ref[...].astype(jnp.float32) * inv)

    return pl.pallas_call(
        body,
        grid=(B // BT,),
        in_specs=[
            pl.BlockSpec((BT, DF), lambda i: (i, 0)),
            pl.BlockSpec((DF, H), lambda i: (0, 0)),
            pl.BlockSpec((1, H), lambda i: (0, 0)),
            pl.BlockSpec((BT, 1), lambda i: (i, 0)),
            pl.BlockSpec((BT, H), lambda i: (i, 0)),
            pl.BlockSpec((BT, H), lambda i: (i, 0)),
            pl.BlockSpec((BT, 1), lambda i: (i, 0)),
        ],
        out_specs=pl.BlockSpec((BT, H), lambda i: (i, 0)),
        out_shape=jax.ShapeDtypeStruct((B, H), jnp.float32),
    )(float_feat, W, b_row, len_col, tsum, crow2, cid_col)


def kernel(cat_feat, float_feat, title, title_len, emb_cat, W_float, b_float,
           emb_text):
    title_flat = title.astype(jnp.int32).reshape(-1)
    cat_ids = cat_feat.astype(jnp.int32)
    ecat_rows = _tc_transpose_table(emb_cat.T,
                                    jnp.eye(H, dtype=jnp.bfloat16))
    tsum = _sc_bag(title_flat, emb_text.astype(jnp.bfloat16))
    crow2 = _sc_cat(cat_ids, ecat_rows)
    return _tc_combine(float_feat, W_float, b_float.reshape(1, H),
                       title_len.astype(jnp.int32).reshape(B, 1), tsum,
                       crow2, cat_ids.reshape(B, 1))


# speedup vs baseline: 3.0035x; 1.0051x over previous
import functools

import jax
import jax.numpy as jnp
from jax import lax
from jax.experimental import pallas as pl
from jax.experimental.pallas import tpu as pltpu
from jax.experimental.pallas import tpu_sc as plsc

B = 16384
L = 50
DF = 128
H = 64
NC, NS = 2, 16
NW = NC * NS
BPW = B // NW
CB = 8
NCHUNK = BPW // CB
IPC = CB * L
GW = 80
NG = IPC // GW
VL = 16
NH = H // VL
VOCAB_QUADS = 250000


def _sc_bag(title_flat, emb_text):
    mesh = plsc.VectorSubcoreMesh(core_axis_name="c", subcore_axis_name="s")
    ILV = plsc.PackFormat.INTERLEAVED

    @functools.partial(
        pl.kernel,
        out_type=jax.ShapeDtypeStruct((B, H), jnp.bfloat16),
        mesh=mesh,
        compiler_params=pltpu.CompilerParams(use_tc_tiling_on_sc=False,
                                             needs_layout_passes=False),
        scratch_types=[
            pltpu.VMEM((IPC,), jnp.int32),
            pltpu.VMEM((IPC, H), jnp.bfloat16),
            pltpu.VMEM((CB, H), jnp.bfloat16),
            pltpu.SemaphoreType.DMA,
        ],
    )
    def k(title_hbm, etext_hbm, tsum_hbm, idx_v, rows_v, out_v, sem):
        wid = lax.axis_index("s") * NC + lax.axis_index("c")

        def unp(r, u):
            return plsc.unpack(rows_v[r, pl.ds(u * 32, 32)], format=ILV)

        def chunk_body(c, carry):
            b0 = wid * BPW + c * CB
            pltpu.sync_copy(title_hbm.at[pl.ds(b0 * L, IPC)], idx_v)
            copies = [
                pltpu.async_copy(etext_hbm.at[idx_v.at[pl.ds(g * GW, GW)]],
                                 rows_v.at[pl.ds(g * GW, GW), :], sem)
                for g in range(NG)
            ]
            for cp in copies:
                cp.wait()

            for b in range(CB):
                r0 = b * L
                acc = list(unp(r0, 0) + unp(r0, 1))
                for j in range(1, L):
                    x = unp(r0 + j, 0) + unp(r0 + j, 1)
                    for q in range(4):
                        acc[q] = acc[q] + x[q]
                out_v[b, pl.ds(0, 32)] = plsc.pack(acc[0], acc[1], format=ILV)
                out_v[b, pl.ds(32, 32)] = plsc.pack(acc[2], acc[3], format=ILV)
            pltpu.sync_copy(out_v, tsum_hbm.at[pl.ds(b0, CB), :])
            return carry

        lax.fori_loop(0, NCHUNK, chunk_body, 0)

    return k(title_flat, emb_text)


def _sc_cat(cat_ids, emb_cat):
    NBUF = 8
    NGRP = BPW // NBUF

    mesh = plsc.VectorSubcoreMesh(core_axis_name="c", subcore_axis_name="s")

    @functools.partial(
        pl.kernel,
        out_type=jax.ShapeDtypeStruct((B, H), jnp.float32),
        mesh=mesh,
        scratch_types=[
            pltpu.VMEM((BPW,), jnp.int32),
            pltpu.VMEM((2, NBUF, 8, H), jnp.float32),
            pltpu.VMEM((NBUF, H), jnp.float32),
            pltpu.SemaphoreType.DMA,
            [pltpu.SemaphoreType.DMA] * (2 * NBUF),
        ],
    )
    def k(cat_hbm, ecat_hbm, crow_hbm, idx_v, tiles_v, out_v, isem, sems):
        wid = lax.axis_index("s") * NC + lax.axis_index("c")
        i0 = wid * BPW
        pltpu.async_copy(cat_hbm.at[pl.ds(i0, BPW)], idx_v, isem).wait()

        def load_ids(t):
            return idx_v[pl.ds(t * 2 * NBUF, 2 * NBUF)]

        def fire_group(ids, bank):
            for p in range(NBUF):
                tid = (ids[bank * NBUF + p] // 8) * 8
                pltpu.async_copy(
                    ecat_hbm.at[pl.ds(pl.multiple_of(tid, 8), 8), :],
                    tiles_v.at[bank, p], sems[bank * NBUF + p])

        def drain_group(ids, g, bank):
            for p in range(NBUF):
                pltpu.make_async_copy(ecat_hbm.at[pl.ds(0, 8), :],
                                      tiles_v.at[bank, p],
                                      sems[bank * NBUF + p]).wait()
                r = lax.rem(ids[bank * NBUF + p], 8)
                for h in range(NH):
                    out_v[p, pl.ds(h * VL, VL)] = \
                        tiles_v[bank, p, r, pl.ds(h * VL, VL)]
            pltpu.sync_copy(out_v,
                            crow_hbm.at[pl.ds(i0 + g * NBUF, NBUF), :])

        fire_group(load_ids(0), 0)

        def body(t, _):
            g = 2 * t
            ids = load_ids(t)
            fire_group(ids, 1)
            drain_group(ids, g, 0)

            @pl.when(t + 1 < NGRP // 2)
            def _():
                fire_group(load_ids(t + 1), 0)

            drain_group(ids, g + 1, 1)
            return 0

        lax.fori_loop(0, NGRP // 2, body, 0)

    return k(cat_ids, emb_cat)


TB = 16384
NTB = (1000001 + TB - 1) // TB


def _tc_transpose_table(ecat_t, ident):
    def body(t_ref, i_ref, o_ref):
        blk = t_ref[...].astype(jnp.bfloat16)
        o_ref[...] = lax.dot_general(blk, i_ref[...],
                                     (((0,), (0,)), ((), ())),
                                     preferred_element_type=jnp.float32)

    return pl.pallas_call(
        body,
        grid=(NTB,),
        in_specs=[
            pl.BlockSpec((H, TB), lambda i: (0, i)),
            pl.BlockSpec((H, H), lambda i: (0, 0)),
        ],
        out_specs=pl.BlockSpec((TB, H), lambda i: (i, 0)),
        out_shape=jax.ShapeDtypeStruct((NTB * TB, H), jnp.float32),
    )(ecat_t, ident)


def _tc_combine(float_feat, W, b_row, len_col, tsum, crow2, cid_col):
    BT = 2048

    def body(ff_ref, w_ref, b_ref, len_ref, ts_ref, cr_ref, cid_ref, o_ref):
        inv = 1.0 / len_ref[...].astype(jnp.float32)
        proj = jnp.dot(ff_ref[...], w_ref[...],
                       preferred_element_type=jnp.float32)
        cat = cr_ref[...]
        o_ref[...] = (cat + proj + b_ref[...]
                      + ts_ref[...].astype(jnp.float32) * inv)

    return pl.pallas_call(
        body,
        grid=(B // BT,),
        in_specs=[
            pl.BlockSpec((BT, DF), lambda i: (i, 0)),
            pl.BlockSpec((DF, H), lambda i: (0, 0)),
            pl.BlockSpec((1, H), lambda i: (0, 0)),
            pl.BlockSpec((BT, 1), lambda i: (i, 0)),
            pl.BlockSpec((BT, H), lambda i: (i, 0)),
            pl.BlockSpec((BT, H), lambda i: (i, 0)),
            pl.BlockSpec((BT, 1), lambda i: (i, 0)),
        ],
        out_specs=pl.BlockSpec((BT, H), lambda i: (i, 0)),
        out_shape=jax.ShapeDtypeStruct((B, H), jnp.float32),
    )(float_feat, W, b_row, len_col, tsum, crow2, cid_col)


def kernel(cat_feat, float_feat, title, title_len, emb_cat, W_float, b_float,
           emb_text):
    title_flat = title.astype(jnp.int32).reshape(-1)
    cat_ids = cat_feat.astype(jnp.int32)
    ecat_rows = _tc_transpose_table(emb_cat.T,
                                    jnp.eye(H, dtype=jnp.bfloat16))
    tsum = _sc_bag(title_flat, emb_text.astype(jnp.bfloat16))
    crow2 = _sc_cat(cat_ids, ecat_rows)
    return _tc_combine(float_feat, W_float, b_float.reshape(1, H),
                       title_len.astype(jnp.int32).reshape(B, 1), tsum,
                       crow2, cat_ids.reshape(B, 1))


# speedup vs baseline: 3.0207x; 1.0057x over previous
import functools

import jax
import jax.numpy as jnp
from jax import lax
from jax.experimental import pallas as pl
from jax.experimental.pallas import tpu as pltpu
from jax.experimental.pallas import tpu_sc as plsc

B = 16384
L = 50
DF = 128
H = 64
NC, NS = 2, 16
NW = NC * NS
BPW = B // NW
CB = 8
NCHUNK = BPW // CB
IPC = CB * L
GW = 80
NG = IPC // GW
VL = 16
NH = H // VL
VOCAB_QUADS = 250000


def _sc_bag(title_flat, emb_text):
    mesh = plsc.VectorSubcoreMesh(core_axis_name="c", subcore_axis_name="s")
    ILV = plsc.PackFormat.INTERLEAVED

    @functools.partial(
        pl.kernel,
        out_type=jax.ShapeDtypeStruct((B, H), jnp.bfloat16),
        mesh=mesh,
        compiler_params=pltpu.CompilerParams(use_tc_tiling_on_sc=False,
                                             needs_layout_passes=False),
        scratch_types=[
            pltpu.VMEM((IPC,), jnp.int32),
            pltpu.VMEM((IPC, H), jnp.bfloat16),
            pltpu.VMEM((CB, H), jnp.bfloat16),
            pltpu.SemaphoreType.DMA,
        ],
    )
    def k(title_hbm, etext_hbm, tsum_hbm, idx_v, rows_v, out_v, sem):
        wid = lax.axis_index("s") * NC + lax.axis_index("c")

        def unp(r, u):
            return plsc.unpack(rows_v[r, pl.ds(u * 32, 32)], format=ILV)

        def chunk_body(c, carry):
            b0 = wid * BPW + c * CB
            pltpu.sync_copy(title_hbm.at[pl.ds(b0 * L, IPC)], idx_v)
            copies = [
                pltpu.async_copy(etext_hbm.at[idx_v.at[pl.ds(g * GW, GW)]],
                                 rows_v.at[pl.ds(g * GW, GW), :], sem)
                for g in range(NG)
            ]
            for cp in copies:
                cp.wait()

            for b in range(CB):
                r0 = b * L
                acc = list(unp(r0, 0) + unp(r0, 1))
                for j in range(1, L):
                    x = unp(r0 + j, 0) + unp(r0 + j, 1)
                    for q in range(4):
                        acc[q] = acc[q] + x[q]
                out_v[b, pl.ds(0, 32)] = plsc.pack(acc[0], acc[1], format=ILV)
                out_v[b, pl.ds(32, 32)] = plsc.pack(acc[2], acc[3], format=ILV)
            pltpu.sync_copy(out_v, tsum_hbm.at[pl.ds(b0, CB), :])
            return carry

        lax.fori_loop(0, NCHUNK, chunk_body, 0)

    return k(title_flat, emb_text)


def _sc_cat(cat_ids, emb_cat):
    NBUF = 8
    NGRP = BPW // NBUF

    mesh = plsc.VectorSubcoreMesh(core_axis_name="c", subcore_axis_name="s")

    @functools.partial(
        pl.kernel,
        out_type=jax.ShapeDtypeStruct((B, H), jnp.float32),
        mesh=mesh,
        scratch_types=[
            pltpu.VMEM((BPW,), jnp.int32),
            pltpu.VMEM((2, NBUF, 8, H), jnp.float32),
            pltpu.VMEM((2, NBUF, H), jnp.float32),
            pltpu.SemaphoreType.DMA,
            [pltpu.SemaphoreType.DMA] * (2 * NBUF),
            [pltpu.SemaphoreType.DMA] * 2,
        ],
    )
    def k(cat_hbm, ecat_hbm, crow_hbm, idx_v, tiles_v, out_v, isem, sems,
          osems):
        wid = lax.axis_index("s") * NC + lax.axis_index("c")
        i0 = wid * BPW
        pltpu.async_copy(cat_hbm.at[pl.ds(i0, BPW)], idx_v, isem).wait()

        def load_ids(t):
            return idx_v[pl.ds(t * 2 * NBUF, 2 * NBUF)]

        def fire_group(ids, bank):
            for p in range(NBUF):
                tid = (ids[bank * NBUF + p] // 8) * 8
                pltpu.async_copy(
                    ecat_hbm.at[pl.ds(pl.multiple_of(tid, 8), 8), :],
                    tiles_v.at[bank, p], sems[bank * NBUF + p])

        def drain_group(ids, g, bank):
            @pl.when(g >= 2)
            def _():
                pltpu.make_async_copy(
                    out_v.at[bank],
                    crow_hbm.at[pl.ds(i0, NBUF), :], osems[bank]).wait()

            for p in range(NBUF):
                pltpu.make_async_copy(ecat_hbm.at[pl.ds(0, 8), :],
                                      tiles_v.at[bank, p],
                                      sems[bank * NBUF + p]).wait()
                r = lax.rem(ids[bank * NBUF + p], 8)
                for h in range(NH):
                    out_v[bank, p, pl.ds(h * VL, VL)] = \
                        tiles_v[bank, p, r, pl.ds(h * VL, VL)]
            pltpu.async_copy(out_v.at[bank],
                             crow_hbm.at[pl.ds(i0 + g * NBUF, NBUF), :],
                             osems[bank])

        fire_group(load_ids(0), 0)

        def body(t, _):
            g = 2 * t
            ids = load_ids(t)
            fire_group(ids, 1)
            drain_group(ids, g, 0)

            @pl.when(t + 1 < NGRP // 2)
            def _():
                fire_group(load_ids(t + 1), 0)

            drain_group(ids, g + 1, 1)
            return 0

        lax.fori_loop(0, NGRP // 2, body, 0)
        for bank in range(2):
            pltpu.make_async_copy(out_v.at[bank],
                                  crow_hbm.at[pl.ds(i0, NBUF), :],
                                  osems[bank]).wait()

    return k(cat_ids, emb_cat)


TB = 16384
NTB = (1000001 + TB - 1) // TB


def _tc_transpose_table(ecat_t, ident):
    def body(t_ref, i_ref, o_ref):
        blk = t_ref[...].astype(jnp.bfloat16)
        o_ref[...] = lax.dot_general(blk, i_ref[...],
                                     (((0,), (0,)), ((), ())),
                                     preferred_element_type=jnp.float32)

    return pl.pallas_call(
        body,
        grid=(NTB,),
        in_specs=[
            pl.BlockSpec((H, TB), lambda i: (0, i)),
            pl.BlockSpec((H, H), lambda i: (0, 0)),
        ],
        out_specs=pl.BlockSpec((TB, H), lambda i: (i, 0)),
        out_shape=jax.ShapeDtypeStruct((NTB * TB, H), jnp.float32),
    )(ecat_t, ident)


def _tc_combine(float_feat, W, b_row, len_col, tsum, crow2, cid_col):
    BT = 2048

    def body(ff_ref, w_ref, b_ref, len_ref, ts_ref, cr_ref, cid_ref, o_ref):
        inv = 1.0 / len_ref[...].astype(jnp.float32)
        proj = jnp.dot(ff_ref[...], w_ref[...],
                       preferred_element_type=jnp.float32)
        cat = cr_ref[...]
        o_ref[...] = (cat + proj + b_ref[...]
                      + ts_ref[...].astype(jnp.float32) * inv)

    return pl.pallas_call(
        body,
        grid=(B // BT,),
        in_specs=[
            pl.BlockSpec((BT, DF), lambda i: (i, 0)),
            pl.BlockSpec((DF, H), lambda i: (0, 0)),
            pl.BlockSpec((1, H), lambda i: (0, 0)),
            pl.BlockSpec((BT, 1), lambda i: (i, 0)),
            pl.BlockSpec((BT, H), lambda i: (i, 0)),
            pl.BlockSpec((BT, H), lambda i: (i, 0)),
            pl.BlockSpec((BT, 1), lambda i: (i, 0)),
        ],
        out_specs=pl.BlockSpec((BT, H), lambda i: (i, 0)),
        out_shape=jax.ShapeDtypeStruct((B, H), jnp.float32),
    )(float_feat, W, b_row, len_col, tsum, crow2, cid_col)


def kernel(cat_feat, float_feat, title, title_len, emb_cat, W_float, b_float,
           emb_text):
    title_flat = title.astype(jnp.int32).reshape(-1)
    cat_ids = cat_feat.astype(jnp.int32)
    ecat_rows = _tc_transpose_table(emb_cat.T,
                                    jnp.eye(H, dtype=jnp.bfloat16))
    tsum = _sc_bag(title_flat, emb_text.astype(jnp.bfloat16))
    crow2 = _sc_cat(cat_ids, ecat_rows)
    return _tc_combine(float_feat, W_float, b_float.reshape(1, H),
                       title_len.astype(jnp.int32).reshape(B, 1), tsum,
                       crow2, cat_ids.reshape(B, 1))
